# baseline (speedup 1.0000x reference)
import functools

import jax
import jax.numpy as jnp
from jax import lax
from jax.experimental import pallas as pl
from jax.experimental.pallas import tpu as pltpu
from jax.experimental.pallas import tpu_sc as plsc

NC = 2
NS = 16
NW = NC * NS
K = 128


def _sc_aggregate(x, sd, ea2d, n_acc, chunks):
  n, d = x.shape
  de = 16
  iters2 = chunks // (2 * NW)
  erows = K * de // 128

  mesh = plsc.VectorSubcoreMesh(core_axis_name="c", subcore_axis_name="s")

  @functools.partial(
      pl.kernel,
      mesh=mesh,
      out_type=[
          jax.ShapeDtypeStruct((NC, n, d), jnp.float32),
          jax.ShapeDtypeStruct((NC, n, d), jnp.float32),
      ],
      scratch_types=[
          pltpu.VMEM_SHARED((n_acc, d), jnp.float32),
          pltpu.VMEM((2 * K,), jnp.int32),
          pltpu.VMEM((2 * K,), jnp.int32),
          pltpu.VMEM((K,), jnp.int32),
          pltpu.VMEM((K,), jnp.int32),
          pltpu.VMEM((K, d), jnp.float32),
          pltpu.VMEM((K, d), jnp.float32),
          pltpu.VMEM((erows, d), jnp.float32),
          pltpu.VMEM((erows, d), jnp.float32),
          pltpu.SemaphoreType.DMA,
          pltpu.SemaphoreType.DMA,
      ],
  )
  def agg_kernel(x_hbm, sd_hbm, ea_hbm,
                 aggx_hbm, agge_hbm,
                 acc_s, sd_a, sd_b, dst_a, dst_b, rows_a, rows_b,
                 ea_a, ea_b, sem_a, sem_b):
    cid = lax.axis_index("c")
    sid = lax.axis_index("s")
    wid = sid * NC + cid

    def zero_buf(buf):
      @pl.loop(0, K)
      def _(i):
        for j in range(d // 16):
          buf[i, pl.ds(j * 16, 16)] = jnp.zeros((16,), jnp.float32)

    nblk = n_acc // K

    def zero_acc():
      @pl.loop(0, (nblk + NS - 1) // NS)
      def _(t):
        b = sid + t * NS

        @pl.when(b < nblk)
        def _():
          pltpu.sync_copy(rows_a, acc_s.at[pl.ds(b * K, K)])

    wb_full = n // K
    wb_tail = n - wb_full * K

    def writeback(out_hbm):
      @pl.loop(0, (wb_full + NS - 1) // NS)
      def _(t):
        b = sid + t * NS

        @pl.when(b < wb_full)
        def _():
          pltpu.sync_copy(acc_s.at[pl.ds(b * K, K)], rows_b)
          pltpu.sync_copy(rows_b, out_hbm.at[cid].at[pl.ds(b * K, K)])

      if wb_tail:
        @pl.when(sid == 0)
        def _():
          pltpu.sync_copy(acc_s.at[pl.ds(wb_full * K, wb_tail)],
                          rows_b.at[pl.ds(0, wb_tail)])
          pltpu.sync_copy(rows_b.at[pl.ds(0, wb_tail)],
                          out_hbm.at[cid].at[pl.ds(wb_full * K, wb_tail)])

    def load_dst(sd_v, dst_v):
      for v in range(K // 16):
        dst_v[pl.ds(v * 16, 16)] = sd_v[pl.ds(K + v * 16, 16)]

    zero_buf(rows_a)
    zero_acc()
    plsc.subcore_barrier()

    @pl.loop(0, iters2)
    def _(it):
      ci0 = wid + (2 * it) * NW
      ci1 = ci0 + NW
      pltpu.sync_copy(sd_hbm.at[pl.ds(ci0 * 2 * K, 2 * K)], sd_a)
      cp_a = pltpu.async_copy(x_hbm.at[sd_a.at[pl.ds(0, K)]], rows_a, sem_a)
      load_dst(sd_a, dst_a)
      pltpu.sync_copy(sd_hbm.at[pl.ds(ci1 * 2 * K, 2 * K)], sd_b)
      cp_b = pltpu.async_copy(x_hbm.at[sd_b.at[pl.ds(0, K)]], rows_b, sem_b)
      load_dst(sd_b, dst_b)
      cp_a.wait()
      pltpu.sync_copy(rows_a, acc_s.at[dst_a], add=True)
      cp_b.wait()
      pltpu.sync_copy(rows_b, acc_s.at[dst_b], add=True)

    plsc.subcore_barrier()
    writeback(aggx_hbm)
    plsc.subcore_barrier()

    zero_buf(rows_a)
    zero_acc()
    plsc.subcore_barrier()
    zero_buf(rows_b)

    @pl.loop(0, K)
    def _(i):
      rows_a[i, pl.ds(16, 16)] = jnp.ones((16,), jnp.float32)
      rows_b[i, pl.ds(16, 16)] = jnp.ones((16,), jnp.float32)

    def unpack(ea_v, rows):
      for i in range(erows):
        for jj in range(8):
          rows[i * 8 + jj, pl.ds(0, 16)] = ea_v[i, pl.ds(jj * 16, 16)]

    @pl.loop(0, iters2)
    def _(it):
      ci0 = wid + (2 * it) * NW
      ci1 = ci0 + NW
      pltpu.sync_copy(sd_hbm.at[pl.ds(ci0 * 2 * K, 2 * K)], sd_a)
      eoff0 = pl.multiple_of(ci0 * erows, 8)
      cp_a = pltpu.async_copy(ea_hbm.at[pl.ds(eoff0, erows)], ea_a, sem_a)
      load_dst(sd_a, dst_a)
      pltpu.sync_copy(sd_hbm.at[pl.ds(ci1 * 2 * K, 2 * K)], sd_b)
      eoff1 = pl.multiple_of(ci1 * erows, 8)
      cp_b = pltpu.async_copy(ea_hbm.at[pl.ds(eoff1, erows)], ea_b, sem_b)
      load_dst(sd_b, dst_b)
      cp_a.wait()
      unpack(ea_a, rows_a)
      pltpu.sync_copy(rows_a, acc_s.at[dst_a], add=True)
      cp_b.wait()
      unpack(ea_b, rows_b)
      pltpu.sync_copy(rows_b, acc_s.at[dst_b], add=True)

    plsc.subcore_barrier()
    writeback(agge_hbm)

  return agg_kernel(x, sd, ea2d)


def _tc_mlp(aggx, agge, x, lin_W, lin_b, W1, b1, W2, b2):
  n, d = x.shape
  de = lin_W.shape[0]
  bn = 2000
  grid = (n // bn,)

  def body(aggx_ref, agge_ref, x_ref, linw_ref, linb_ref,
           w1_ref, b1_ref, w2_ref, b2_ref, out_ref):
    ax = aggx_ref[0] + aggx_ref[1]
    ag = agge_ref[0] + agge_ref[1]
    ae = ag[:, 0:de]
    dg = ag[:, de:de + 1]
    ev = jnp.dot(ae, linw_ref[...], preferred_element_type=jnp.float32)
    ev = ev + dg * linb_ref[...]
    out = ax + ev + x_ref[...]
    h = jnp.maximum(
        jnp.dot(out, w1_ref[...], preferred_element_type=jnp.float32)
        + b1_ref[...], 0.0)
    out_ref[...] = (
        jnp.dot(h, w2_ref[...], preferred_element_type=jnp.float32)
        + b2_ref[...])

  return pl.pallas_call(
      body,
      grid=grid,
      in_specs=[
          pl.BlockSpec((NC, bn, d), lambda i: (0, i, 0)),
          pl.BlockSpec((NC, bn, d), lambda i: (0, i, 0)),
          pl.BlockSpec((bn, d), lambda i: (i, 0)),
          pl.BlockSpec((de, d), lambda i: (0, 0)),
          pl.BlockSpec((1, d), lambda i: (0, 0)),
          pl.BlockSpec((d, d), lambda i: (0, 0)),
          pl.BlockSpec((1, d), lambda i: (0, 0)),
          pl.BlockSpec((d, d), lambda i: (0, 0)),
          pl.BlockSpec((1, d), lambda i: (0, 0)),
      ],
      out_specs=pl.BlockSpec((bn, d), lambda i: (i, 0)),
      out_shape=jax.ShapeDtypeStruct((n, d), jnp.float32),
  )(aggx, agge, x, lin_W, lin_b.reshape(1, d), W1, b1.reshape(1, d),
    W2, b2.reshape(1, d))


def kernel(x, edge_index, edge_attr, lin_W, lin_b, W1, b1, W2, b2):
  n = x.shape[0]
  e, de = edge_attr.shape
  src = edge_index[0].astype(jnp.int32)
  dst = edge_index[1].astype(jnp.int32)

  ept = 2 * NW * K
  e_pad = (e + ept - 1) // ept * ept
  n_acc = (n // K + 1) * K
  if e_pad != e:
    pad = e_pad - e
    src = jnp.concatenate([src, jnp.zeros((pad,), jnp.int32)])
    dst = jnp.concatenate([dst, jnp.full((pad,), n_acc - 1, jnp.int32)])
    edge_attr = jnp.concatenate(
        [edge_attr, jnp.zeros((pad, de), edge_attr.dtype)])

  chunks = e_pad // K
  sd = jnp.stack([src.reshape(chunks, K), dst.reshape(chunks, K)],
                 axis=1).reshape(-1)
  ea2d = edge_attr.reshape(e_pad * de // 128, 128)

  aggx, agge = _sc_aggregate(x, sd, ea2d, n_acc, chunks)
  return _tc_mlp(aggx, agge, x, lin_W, lin_b, W1, b1, W2, b2)


# speedup vs baseline: 1.6506x; 1.6506x over previous
import functools

import jax
import jax.numpy as jnp
from jax import lax
from jax.experimental import pallas as pl
from jax.experimental.pallas import tpu as pltpu
from jax.experimental.pallas import tpu_sc as plsc

NC = 2
NS = 16
NW = NC * NS
K = 128


def _sc_aggregate(x, src, dst, ea_packed):
  n, d = x.shape
  e = src.shape[0]
  chunks = e // K
  iters = (chunks + NW - 1) // NW
  rpt = n // NS

  mesh = plsc.VectorSubcoreMesh(core_axis_name="c", subcore_axis_name="s")

  @functools.partial(
      pl.kernel,
      mesh=mesh,
      out_type=[
          jax.ShapeDtypeStruct((NC, n, d), jnp.float32),
          jax.ShapeDtypeStruct((NC, n, d), jnp.float32),
      ],
      scratch_types=[
          pltpu.VMEM_SHARED((n, d), jnp.float32),
          pltpu.VMEM((K,), jnp.int32),
          pltpu.VMEM((K,), jnp.int32),
          pltpu.VMEM((K,), jnp.int32),
          pltpu.VMEM((K,), jnp.int32),
          pltpu.VMEM((K, d), jnp.float32),
          pltpu.VMEM((K, d), jnp.float32),
          pltpu.VMEM((K // 8, d), jnp.float32),
          pltpu.SemaphoreType.DMA,
          pltpu.SemaphoreType.DMA,
      ],
  )
  def agg_kernel(x_hbm, src_hbm, dst_hbm, ea_hbm,
                 aggx_hbm, agge_hbm,
                 acc_s, src_v, dst_v, src_w, dst_w, rows_v, rows_w,
                 eapack_v, sem, sem2):
    cid = lax.axis_index("c")
    sid = lax.axis_index("s")
    wid = sid * NC + cid

    def zero_rows_v():
      @pl.loop(0, K)
      def _(i):
        for j in range(d // 16):
          rows_v[i, pl.ds(j * 16, 16)] = jnp.zeros((16,), jnp.float32)

    def zero_acc():
      full = rpt // K * K
      @pl.loop(0, full, step=K)
      def _(r):
        pltpu.sync_copy(rows_v, acc_s.at[pl.ds(sid * rpt + r, K)])
      rem = rpt % K
      if rem:
        pltpu.sync_copy(rows_v.at[pl.ds(0, rem)],
                        acc_s.at[pl.ds(sid * rpt + full, rem)])

    wb = rpt // 8 * 8
    lastn = n - wb * (NS - 1)
    wbase = sid * wb
    nwb = lax.select(sid == NS - 1, lastn // 16, wb // 16)

    def writeback(out_hbm):
      @pl.loop(0, nwb)
      def _(t):
        r = wbase + t * 16
        pltpu.sync_copy(acc_s.at[pl.ds(r, 16)], rows_v.at[pl.ds(0, 16)])
        pltpu.sync_copy(rows_v.at[pl.ds(0, 16)],
                        out_hbm.at[cid].at[pl.ds(r, 16)])

    zero_rows_v()
    zero_acc()
    plsc.subcore_barrier()

    @pl.loop(0, iters // 2)
    def _(it):
      ci0 = wid + (2 * it) * NW
      ci1 = ci0 + NW
      pltpu.sync_copy(src_hbm.at[pl.ds(ci0 * K, K)], src_v)
      pltpu.sync_copy(dst_hbm.at[pl.ds(ci0 * K, K)], dst_v)
      cp_a = pltpu.async_copy(x_hbm.at[src_v], rows_v, sem)
      pltpu.sync_copy(src_hbm.at[pl.ds(ci1 * K, K)], src_w)
      pltpu.sync_copy(dst_hbm.at[pl.ds(ci1 * K, K)], dst_w)
      cp_b = pltpu.async_copy(x_hbm.at[src_w], rows_w, sem2)
      cp_a.wait()
      pltpu.sync_copy(rows_v, acc_s.at[dst_v], add=True)
      cp_b.wait()
      pltpu.sync_copy(rows_w, acc_s.at[dst_w], add=True)

    if iters % 2:
      ci = wid + (iters - 1) * NW

      @pl.when(ci < chunks)
      def _():
        pltpu.sync_copy(src_hbm.at[pl.ds(ci * K, K)], src_v)
        pltpu.sync_copy(dst_hbm.at[pl.ds(ci * K, K)], dst_v)
        pltpu.async_copy(x_hbm.at[src_v], rows_v, sem).wait()
        pltpu.sync_copy(rows_v, acc_s.at[dst_v], add=True)

    plsc.subcore_barrier()
    writeback(aggx_hbm)
    plsc.subcore_barrier()

    zero_rows_v()
    zero_acc()
    plsc.subcore_barrier()

    @pl.loop(0, K)
    def _(i):
      rows_v[i, pl.ds(16, 16)] = jnp.ones((16,), jnp.float32)

    @pl.loop(0, iters)
    def _(it):
      ci = wid + it * NW

      @pl.when(ci < chunks)
      def _():
        base = ci * K
        pltpu.sync_copy(dst_hbm.at[pl.ds(base, K)], dst_v)
        off8 = pl.multiple_of(ci * (K // 8), 8)
        pltpu.sync_copy(ea_hbm.at[pl.ds(off8, K // 8)], eapack_v)
        for i in range(K // 8):
          for j in range(8):
            rows_v[i * 8 + j, pl.ds(0, 16)] = eapack_v[i, pl.ds(j * 16, 16)]
        pltpu.sync_copy(rows_v, acc_s.at[dst_v], add=True)

    plsc.subcore_barrier()
    writeback(agge_hbm)

  return agg_kernel(x, src, dst, ea_packed)


def _tc_mlp(aggx, agge, x, lin_W, lin_b, W1, b1, W2, b2):
  n, d = x.shape
  de = lin_W.shape[0]
  bn = 2000
  grid = (n // bn,)

  def body(aggx_ref, agge_ref, x_ref, linw_ref, linb_ref,
           w1_ref, b1_ref, w2_ref, b2_ref, out_ref):
    ax = aggx_ref[0] + aggx_ref[1]
    ag = agge_ref[0] + agge_ref[1]
    ae = ag[:, 0:de]
    dg = ag[:, de:de + 1]
    ev = jnp.dot(ae, linw_ref[...], preferred_element_type=jnp.float32)
    ev = ev + dg * linb_ref[...]
    out = ax + ev + x_ref[...]
    h = jnp.maximum(
        jnp.dot(out, w1_ref[...], preferred_element_type=jnp.float32)
        + b1_ref[...], 0.0)
    out_ref[...] = (
        jnp.dot(h, w2_ref[...], preferred_element_type=jnp.float32)
        + b2_ref[...])

  return pl.pallas_call(
      body,
      grid=grid,
      in_specs=[
          pl.BlockSpec((NC, bn, d), lambda i: (0, i, 0)),
          pl.BlockSpec((NC, bn, d), lambda i: (0, i, 0)),
          pl.BlockSpec((bn, d), lambda i: (i, 0)),
          pl.BlockSpec((de, d), lambda i: (0, 0)),
          pl.BlockSpec((1, d), lambda i: (0, 0)),
          pl.BlockSpec((d, d), lambda i: (0, 0)),
          pl.BlockSpec((1, d), lambda i: (0, 0)),
          pl.BlockSpec((d, d), lambda i: (0, 0)),
          pl.BlockSpec((1, d), lambda i: (0, 0)),
      ],
      out_specs=pl.BlockSpec((bn, d), lambda i: (i, 0)),
      out_shape=jax.ShapeDtypeStruct((n, d), jnp.float32),
  )(aggx, agge, x, lin_W, lin_b.reshape(1, d), W1, b1.reshape(1, d),
    W2, b2.reshape(1, d))


def kernel(x, edge_index, edge_attr, lin_W, lin_b, W1, b1, W2, b2):
  src = edge_index[0].astype(jnp.int32)
  dst = edge_index[1].astype(jnp.int32)
  e, de = edge_attr.shape
  ea_packed = edge_attr.reshape(e * de // 128, 128)
  aggx, agge = _sc_aggregate(x, src, dst, ea_packed)
  return _tc_mlp(aggx, agge, x, lin_W, lin_b, W1, b1, W2, b2)


# speedup vs baseline: 1.6821x; 1.0190x over previous
import functools

import jax
import jax.numpy as jnp
from jax import lax
from jax.experimental import pallas as pl
from jax.experimental.pallas import tpu as pltpu
from jax.experimental.pallas import tpu_sc as plsc

NC = 2
NS = 16
NW = NC * NS
K = 128


def _sc_aggregate(x, src, dst, ea_packed):
  n, d = x.shape
  e = src.shape[0]
  chunks = e // K
  iters = (chunks + NW - 1) // NW
  rpt = n // NS

  mesh = plsc.VectorSubcoreMesh(core_axis_name="c", subcore_axis_name="s")

  @functools.partial(
      pl.kernel,
      mesh=mesh,
      out_type=[
          jax.ShapeDtypeStruct((NC, n, d), jnp.float32),
          jax.ShapeDtypeStruct((NC, n, d), jnp.float32),
      ],
      scratch_types=[
          pltpu.VMEM_SHARED((n, d), jnp.float32),
          pltpu.VMEM((K,), jnp.int32),
          pltpu.VMEM((K,), jnp.int32),
          pltpu.VMEM((K,), jnp.int32),
          pltpu.VMEM((K,), jnp.int32),
          pltpu.VMEM((K, d), jnp.float32),
          pltpu.VMEM((K, d), jnp.float32),
          pltpu.VMEM((K // 8, d), jnp.float32),
          pltpu.VMEM((K // 8, d), jnp.float32),
          pltpu.SemaphoreType.DMA,
          pltpu.SemaphoreType.DMA,
      ],
  )
  def agg_kernel(x_hbm, src_hbm, dst_hbm, ea_hbm,
                 aggx_hbm, agge_hbm,
                 acc_s, src_v, dst_v, src_w, dst_w, rows_v, rows_w,
                 eapack_v, eapack_w, sem, sem2):
    cid = lax.axis_index("c")
    sid = lax.axis_index("s")
    wid = sid * NC + cid

    def zero_rows_v():
      @pl.loop(0, K)
      def _(i):
        for j in range(d // 16):
          rows_v[i, pl.ds(j * 16, 16)] = jnp.zeros((16,), jnp.float32)

    nblk = n // K
    tail = n - nblk * K

    def zero_acc():
      @pl.loop(0, (nblk + NS - 1) // NS)
      def _(t):
        b = sid + t * NS

        @pl.when(b < nblk)
        def _():
          pltpu.sync_copy(rows_v, acc_s.at[pl.ds(b * K, K)])

      if tail:
        @pl.when(sid == NS - 1)
        def _():
          pltpu.sync_copy(rows_v.at[pl.ds(0, tail)],
                          acc_s.at[pl.ds(nblk * K, tail)])

    def writeback(out_hbm):
      @pl.loop(0, (nblk + NS - 1) // NS)
      def _(t):
        b = sid + t * NS

        @pl.when(b < nblk)
        def _():
          pltpu.sync_copy(acc_s.at[pl.ds(b * K, K)], rows_w)
          pltpu.sync_copy(rows_w, out_hbm.at[cid].at[pl.ds(b * K, K)])

      if tail:
        @pl.when(sid == 0)
        def _():
          pltpu.sync_copy(acc_s.at[pl.ds(nblk * K, tail)],
                          rows_w.at[pl.ds(0, tail)])
          pltpu.sync_copy(rows_w.at[pl.ds(0, tail)],
                          out_hbm.at[cid].at[pl.ds(nblk * K, tail)])

    zero_rows_v()
    zero_acc()
    plsc.subcore_barrier()

    @pl.loop(0, iters // 2)
    def _(it):
      ci0 = wid + (2 * it) * NW
      ci1 = ci0 + NW
      pltpu.sync_copy(src_hbm.at[pl.ds(ci0 * K, K)], src_v)
      pltpu.sync_copy(dst_hbm.at[pl.ds(ci0 * K, K)], dst_v)
      cp_a = pltpu.async_copy(x_hbm.at[src_v], rows_v, sem)
      pltpu.sync_copy(src_hbm.at[pl.ds(ci1 * K, K)], src_w)
      pltpu.sync_copy(dst_hbm.at[pl.ds(ci1 * K, K)], dst_w)
      cp_b = pltpu.async_copy(x_hbm.at[src_w], rows_w, sem2)
      cp_a.wait()
      pltpu.sync_copy(rows_v, acc_s.at[dst_v], add=True)
      cp_b.wait()
      pltpu.sync_copy(rows_w, acc_s.at[dst_w], add=True)

    if iters % 2:
      ci = wid + (iters - 1) * NW

      @pl.when(ci < chunks)
      def _():
        pltpu.sync_copy(src_hbm.at[pl.ds(ci * K, K)], src_v)
        pltpu.sync_copy(dst_hbm.at[pl.ds(ci * K, K)], dst_v)
        pltpu.async_copy(x_hbm.at[src_v], rows_v, sem).wait()
        pltpu.sync_copy(rows_v, acc_s.at[dst_v], add=True)

    plsc.subcore_barrier()
    writeback(aggx_hbm)
    plsc.subcore_barrier()

    zero_rows_v()
    zero_acc()
    plsc.subcore_barrier()

    @pl.loop(0, K)
    def _(i):
      rows_v[i, pl.ds(16, 16)] = jnp.ones((16,), jnp.float32)

    @pl.loop(0, iters)
    def _(it):
      ci = wid + it * NW

      @pl.when(ci < chunks)
      def _():
        base = ci * K
        pltpu.sync_copy(dst_hbm.at[pl.ds(base, K)], dst_v)
        off8 = pl.multiple_of(ci * (K // 8), 8)
        pltpu.sync_copy(ea_hbm.at[pl.ds(off8, K // 8)], eapack_v)
        for i in range(K // 8):
          for j in range(8):
            rows_v[i * 8 + j, pl.ds(0, 16)] = eapack_v[i, pl.ds(j * 16, 16)]
        pltpu.sync_copy(rows_v, acc_s.at[dst_v], add=True)

    plsc.subcore_barrier()
    writeback(agge_hbm)

  return agg_kernel(x, src, dst, ea_packed)


def _tc_mlp(aggx, agge, x, lin_W, lin_b, W1, b1, W2, b2):
  n, d = x.shape
  de = lin_W.shape[0]
  bn = 2000
  grid = (n // bn,)

  def body(aggx_ref, agge_ref, x_ref, linw_ref, linb_ref,
           w1_ref, b1_ref, w2_ref, b2_ref, out_ref):
    ax = aggx_ref[0] + aggx_ref[1]
    ag = agge_ref[0] + agge_ref[1]
    ae = ag[:, 0:de]
    dg = ag[:, de:de + 1]
    ev = jnp.dot(ae, linw_ref[...], preferred_element_type=jnp.float32)
    ev = ev + dg * linb_ref[...]
    out = ax + ev + x_ref[...]
    h = jnp.maximum(
        jnp.dot(out, w1_ref[...], preferred_element_type=jnp.float32)
        + b1_ref[...], 0.0)
    out_ref[...] = (
        jnp.dot(h, w2_ref[...], preferred_element_type=jnp.float32)
        + b2_ref[...])

  return pl.pallas_call(
      body,
      grid=grid,
      in_specs=[
          pl.BlockSpec((NC, bn, d), lambda i: (0, i, 0)),
          pl.BlockSpec((NC, bn, d), lambda i: (0, i, 0)),
          pl.BlockSpec((bn, d), lambda i: (i, 0)),
          pl.BlockSpec((de, d), lambda i: (0, 0)),
          pl.BlockSpec((1, d), lambda i: (0, 0)),
          pl.BlockSpec((d, d), lambda i: (0, 0)),
          pl.BlockSpec((1, d), lambda i: (0, 0)),
          pl.BlockSpec((d, d), lambda i: (0, 0)),
          pl.BlockSpec((1, d), lambda i: (0, 0)),
      ],
      out_specs=pl.BlockSpec((bn, d), lambda i: (i, 0)),
      out_shape=jax.ShapeDtypeStruct((n, d), jnp.float32),
  )(aggx, agge, x, lin_W, lin_b.reshape(1, d), W1, b1.reshape(1, d),
    W2, b2.reshape(1, d))


def kernel(x, edge_index, edge_attr, lin_W, lin_b, W1, b1, W2, b2):
  src = edge_index[0].astype(jnp.int32)
  dst = edge_index[1].astype(jnp.int32)
  e, de = edge_attr.shape
  ea_packed = edge_attr.reshape(e * de // 128, 128)
  aggx, agge = _sc_aggregate(x, src, dst, ea_packed)
  return _tc_mlp(aggx, agge, x, lin_W, lin_b, W1, b1, W2, b2)
